# baseline (speedup 1.0000x reference)
import functools

import jax
import jax.numpy as jnp
from jax import lax
from jax.experimental import pallas as pl
from jax.experimental.pallas import tpu as pltpu
from jax.experimental.pallas import tpu_sc as plsc

_B = 4
_N = 5000
_NPAD = 5120
_BLKS = _NPAD // 128
_SEG = _NPAD // 8
_SEGCH = 40
_CAP = _SEG + 16
_MAXDET = 100
_MINCONF = 0.7
_NMS_T = 0.3
_BIG = jnp.int32(1 << 20)


def _prep_kernel(rois_ref, cls_ref, out_ref):
    y1 = rois_ref[:, 0]
    x1 = rois_ref[:, 1]
    y2 = rois_ref[:, 2]
    x2 = rois_ref[:, 3]
    dy = cls_ref[:, 0] * 0.1
    dx = cls_ref[:, 1] * 0.1
    dh = cls_ref[:, 2] * 0.2
    dw = cls_ref[:, 3] * 0.2
    cls_f = cls_ref[:, 4]
    raw_scores = cls_ref[:, 5]

    h = y2 - y1
    w = x2 - x1
    cy = y1 + 0.5 * h + dy * h
    cx = x1 + 0.5 * w + dx * w
    h = h * jnp.exp(dh)
    w = w * jnp.exp(dw)
    ry1 = jnp.clip(cy - 0.5 * h, 0.0, 1.0)
    rx1 = jnp.clip(cx - 0.5 * w, 0.0, 1.0)
    ry2 = jnp.clip((cy - 0.5 * h) + h, 0.0, 1.0)
    rx2 = jnp.clip((cx - 0.5 * w) + w, 0.0, 1.0)

    cls_i = cls_f.astype(jnp.int32)
    keep = (cls_i > 0) & (raw_scores >= _MINCONF)
    scores = jnp.where(keep, raw_scores, -1.0)

    off = cls_f * 4.0
    out_ref[:, 0] = ry1 + off
    out_ref[:, 1] = rx1 + off
    out_ref[:, 2] = ry2 + off
    out_ref[:, 3] = rx2 + off
    out_ref[:, 4] = cls_f
    out_ref[:, 5] = scores


def _top2_update(b1v, b1i, b2v, b2i, scm, posm):
    bet1 = (scm > b1v) | ((scm == b1v) & (posm < b1i))
    c2v = jnp.where(bet1, b1v, scm)
    c2i = jnp.where(bet1, b1i, posm)
    bet2 = (c2v > b2v) | ((c2v == b2v) & (c2i < b2i))
    return (
        jnp.where(bet1, scm, b1v),
        jnp.where(bet1, posm, b1i),
        jnp.where(bet2, c2v, b2v),
        jnp.where(bet2, c2i, b2i),
    )


def _xlane_top2(b1v, b1i, b2v, b2i):
    m1 = jnp.max(b1v)
    i1 = jnp.min(jnp.where(b1v == m1, b1i, _BIG))
    sel = (b1v == m1) & (b1i == i1)
    c2v = jnp.where(sel, b2v, b1v)
    c2i = jnp.where(sel, b2i, b1i)
    m2 = jnp.max(c2v)
    i2 = jnp.min(jnp.where(c2v == m2, c2i, _BIG))
    return m1, i1, m2, i2


def _nms_sc(cand_hbm, out_hbm, *refs):
    seg = refs[0:6]
    cch = refs[6:12]
    msg_v = refs[12]
    slots_v = refs[13]
    det = refs[14:20]
    slots_sh = refs[20]
    c = lax.axis_index("c")
    s = lax.axis_index("s")
    g = s // 8
    slot = s % 8
    batch = c * 2 + g
    row = g * 8 + slot
    iota = lax.iota(jnp.int32, 16)
    neg16 = jnp.full((16,), -1.0, jnp.float32)
    big16 = jnp.full((16,), _BIG)

    for k in range(6):
        pltpu.sync_copy(
            cand_hbm.at[batch, k, pl.ds(slot * _SEG, _SEG)], seg[k]
        )

    def compact_body(j, carry):
        cnt, b1v, b1i, b2v, b2i = carry
        idxv = j * 16 + iota
        sc = plsc.load_gather(seg[5], [idxv])
        m = sc > 0.0
        incl = plsc.cumsum(jnp.where(m, 1, 0))
        pos = cnt + incl - 1
        for k in range(5):
            v = plsc.load_gather(seg[k], [idxv])
            plsc.store_scatter(cch[k], [pos], v, mask=m)
        plsc.store_scatter(cch[5], [pos], sc, mask=m)
        scm = jnp.where(m, sc, -1.0)
        posm = jnp.where(m, pos, big16)
        b1v, b1i, b2v, b2i = _top2_update(b1v, b1i, b2v, b2i, scm, posm)
        return (cnt + jnp.max(incl), b1v, b1i, b2v, b2i)

    cnt, b1v0, b1i0, b2v0, b2i0 = lax.fori_loop(
        0, _SEGCH, compact_body, (jnp.int32(0), neg16, big16, neg16, big16)
    )
    plsc.store_scatter(
        cch[5], [cnt + iota], jnp.full((16,), -1.0, jnp.float32)
    )
    nchunks = (cnt + 15) // 16

    @pl.when(slot == 0)
    def _():
        for k in range(6):
            for j in range(8):
                det[k][pl.ds(j * 16, 16)] = jnp.zeros((16,), jnp.float32)

    def round_cond(carry):
        rnd, di, done_my, done_pub = carry[0], carry[1], carry[2], carry[3]
        return (rnd < _MAXDET + 4) & jnp.logical_not(done_pub)

    def round_body(carry):
        rnd, di, done_my, done_pub, b1v, b1i, b2v, b2i, npos = carry
        m1, i1, m2, i2 = _xlane_top2(b1v, b1i, b2v, b2i)
        iis1 = jnp.full((16,), jnp.minimum(i1, jnp.int32(_CAP - 1)), jnp.int32)
        iis2 = jnp.full((16,), jnp.minimum(i2, jnp.int32(_CAP - 1)), jnp.int32)
        has_more = npos > 2

        msg = jnp.where(iota == 5, m1, 0.0)
        msg = jnp.where(iota == 13, m2, msg)
        msg = jnp.where(iota == 6, jnp.where(done_my, 1.0, 0.0), msg)
        msg = jnp.where(iota == 7, jnp.where(has_more, 1.0, 0.0), msg)
        for k in range(5):
            v1 = plsc.load_gather(cch[k], [iis1])
            msg = jnp.where(iota == k, v1, msg)
            v2 = plsc.load_gather(cch[k], [iis2])
            msg = jnp.where(iota == k + 8, v2, msg)
        msg_v[...] = msg
        parity = rnd % 2
        pltpu.sync_copy(
            msg_v, slots_sh.at[pl.ds(parity * 256 + row * 16, 16)]
        )
        plsc.subcore_barrier()
        pltpu.sync_copy(slots_sh.at[pl.ds(parity * 256, 256)], slots_v)

        rows = [
            plsc.load_gather(slots_v, [(g * 8 + w) * 16 + iota])
            for w in range(8)
        ]
        my_done_pub = rows[0][6] > 0.5
        orow = plsc.load_gather(slots_v, [(1 - g) * 128 + iota])
        done_pub_new = my_done_pub & (orow[6] > 0.5)
        hm = [rows[w][7] > 0.5 for w in range(8)]

        e_sc = []
        e_y1 = []
        e_x1 = []
        e_y2 = []
        e_x2 = []
        e_cls = []
        for w in range(8):
            for r in range(2):
                b = r * 8
                e_sc.append(rows[w][5 + b])
                e_y1.append(rows[w][b + 0])
                e_x1.append(rows[w][b + 1])
                e_y2.append(rows[w][b + 2])
                e_x2.append(rows[w][b + 3])
                e_cls.append(rows[w][b + 4])

        stopped = done_my
        exhausted = jnp.bool_(False)
        di_r = di
        own1 = jnp.bool_(False)
        own2 = jnp.bool_(False)
        sent = jnp.float32(1e9)
        av_y1 = jnp.full((16,), sent)
        av_x1 = jnp.full((16,), sent)
        av_y2 = jnp.full((16,), sent)
        av_x2 = jnp.full((16,), sent)
        av_ar = jnp.zeros((16,), jnp.float32)
        csw = [jnp.int32(0) for _ in range(8)]
        for k in range(8):
            blocked = jnp.bool_(False)
            for w in range(8):
                blocked = blocked | ((csw[w] >= 2) & hm[w])
            stopped = stopped | blocked

            esc = e_sc[0]
            eid = jnp.int32(0)
            for j in range(1, 16):
                better = e_sc[j] > esc
                esc = jnp.where(better, e_sc[j], esc)
                eid = jnp.where(better, jnp.int32(j), eid)
            has_cand = esc > 0.0
            exhausted = exhausted | (
                jnp.logical_not(stopped) & jnp.logical_not(has_cand)
            )
            consider = jnp.logical_not(stopped) & has_cand & (
                di_r < _MAXDET
            )
            cy1 = e_y1[0]
            cx1 = e_x1[0]
            cy2 = e_y2[0]
            cx2 = e_x2[0]
            ccls = e_cls[0]
            for j in range(1, 16):
                pick = eid == j
                cy1 = jnp.where(pick, e_y1[j], cy1)
                cx1 = jnp.where(pick, e_x1[j], cx1)
                cy2 = jnp.where(pick, e_y2[j], cy2)
                cx2 = jnp.where(pick, e_x2[j], cx2)
                ccls = jnp.where(pick, e_cls[j], ccls)
            eslot = eid // 2
            erank = eid % 2

            cy1v = jnp.full((16,), cy1)
            cx1v = jnp.full((16,), cx1)
            cy2v = jnp.full((16,), cy2)
            cx2v = jnp.full((16,), cx2)
            careav = (cy2v - cy1v) * (cx2v - cx1v)
            yy1 = jnp.maximum(av_y1, cy1v)
            xx1 = jnp.maximum(av_x1, cx1v)
            yy2 = jnp.minimum(av_y2, cy2v)
            xx2 = jnp.minimum(av_x2, cx2v)
            inter = jnp.maximum(yy2 - yy1, 0.0) * jnp.maximum(
                xx2 - xx1, 0.0
            )
            iou = inter / (av_ar + careav - inter + 1e-8)
            conflict = jnp.max(jnp.where(iou > _NMS_T, 1, 0)) > 0
            accept_k = consider & jnp.logical_not(conflict)
            consumed = consider
            own1 = own1 | (accept_k & (eslot == slot) & (erank == 0))
            own2 = own2 | (accept_k & (eslot == slot) & (erank == 1))

            @pl.when(accept_k & (slot == 0))
            def _(di_r=di_r, cy1=cy1, cx1=cx1, cy2=cy2, cx2=cx2,
                  ccls=ccls, esc=esc):
                hot = iota == 0
                dlane = jnp.full((16,), di_r, jnp.int32)
                offv = ccls * 4.0
                vals = (cy1 - offv, cx1 - offv, cy2 - offv, cx2 - offv,
                        ccls, esc)
                for kk in range(6):
                    plsc.store_scatter(
                        det[kk], [dlane], jnp.full((16,), vals[kk]),
                        mask=hot,
                    )

            lane_k = (iota == k) & accept_k
            av_y1 = jnp.where(lane_k, cy1v, av_y1)
            av_x1 = jnp.where(lane_k, cx1v, av_x1)
            av_y2 = jnp.where(lane_k, cy2v, av_y2)
            av_x2 = jnp.where(lane_k, cx2v, av_x2)
            av_ar = jnp.where(lane_k, careav, av_ar)
            di_r = di_r + jnp.where(accept_k, 1, 0)
            csw = [
                csw[w]
                + jnp.where(consumed & (eslot == w), 1, 0)
                for w in range(8)
            ]
            e_sc = [
                jnp.where(consumed & (eid == j), jnp.float32(-2.0), e_sc[j])
                for j in range(16)
            ]

        done_my_new = done_my | exhausted | (di_r >= _MAXDET)

        plsc.store_scatter(cch[5], [iis1], neg16, mask=(iota == 0) & own1)
        plsc.store_scatter(cch[5], [iis2], neg16, mask=(iota == 0) & own2)

        wvec = [
            (
                jnp.full((16,), av_y1[k]),
                jnp.full((16,), av_x1[k]),
                jnp.full((16,), av_y2[k]),
                jnp.full((16,), av_x2[k]),
                jnp.full((16,), av_ar[k]),
            )
            for k in range(8)
        ]

        def sweep_body(j, carry2):
            nb1v, nb1i, nb2v, nb2i, cntv = carry2
            idxv = j * 16 + iota
            a0 = plsc.load_gather(cch[0], [idxv])
            a1 = plsc.load_gather(cch[1], [idxv])
            a2 = plsc.load_gather(cch[2], [idxv])
            a3 = plsc.load_gather(cch[3], [idxv])
            sc = plsc.load_gather(cch[5], [idxv])
            area = (a2 - a0) * (a3 - a1)
            for (wy1v, wx1v, wy2v, wx2v, wareav) in wvec:
                yy1 = jnp.maximum(wy1v, a0)
                xx1 = jnp.maximum(wx1v, a1)
                yy2 = jnp.minimum(wy2v, a2)
                xx2 = jnp.minimum(wx2v, a3)
                inter = jnp.maximum(yy2 - yy1, 0.0) * jnp.maximum(
                    xx2 - xx1, 0.0
                )
                iou = inter / (wareav + area - inter + 1e-8)
                sc = jnp.where(iou > _NMS_T, -1.0, sc)
            plsc.store_scatter(cch[5], [idxv], sc)
            nb1v, nb1i, nb2v, nb2i = _top2_update(
                nb1v, nb1i, nb2v, nb2i, sc, idxv
            )
            cntv = cntv + jnp.where(sc > 0.0, 1, 0)
            return nb1v, nb1i, nb2v, nb2i, cntv

        nb1v, nb1i, nb2v, nb2i, cntv = lax.fori_loop(
            0,
            nchunks,
            sweep_body,
            (neg16, big16, neg16, big16, jnp.zeros((16,), jnp.int32)),
        )
        npos_new = jnp.sum(cntv)

        return (
            rnd + 1,
            di_r,
            done_my_new,
            done_pub_new,
            nb1v,
            nb1i,
            nb2v,
            nb2i,
            npos_new,
        )

    lax.while_loop(
        round_cond,
        round_body,
        (
            jnp.int32(0),
            jnp.int32(0),
            jnp.bool_(False),
            jnp.bool_(False),
            b1v0,
            b1i0,
            b2v0,
            b2i0,
            cnt,
        ),
    )

    @pl.when(slot == 0)
    def _():
        for k in range(6):
            pltpu.sync_copy(det[k], out_hbm.at[batch, k])


def kernel(rois, classifications):
    rois_t = jnp.transpose(rois, (0, 2, 1))
    cls_t = jnp.transpose(classifications, (0, 2, 1))
    pad = _NPAD - _N
    rois_t = jnp.pad(rois_t, ((0, 0), (0, 0), (0, pad)))
    cls_t = jnp.pad(cls_t, ((0, 0), (0, 0), (0, pad)))
    rois_t = rois_t.reshape(_B, 4, _BLKS, 128)
    cls_t = cls_t.reshape(_B, 6, _BLKS, 128)

    cand = pl.pallas_call(
        _prep_kernel,
        out_shape=jax.ShapeDtypeStruct((_B, 6, _BLKS, 128), jnp.float32),
    )(rois_t, cls_t)
    cand = cand.reshape(_B, 6, _NPAD)

    mesh = plsc.VectorSubcoreMesh(core_axis_name="c", subcore_axis_name="s")
    scratch = (
        [pltpu.VMEM((_SEG,), jnp.float32) for _ in range(6)]
        + [pltpu.VMEM((_CAP,), jnp.float32) for _ in range(6)]
        + [pltpu.VMEM((16,), jnp.float32)]
        + [pltpu.VMEM((256,), jnp.float32)]
        + [pltpu.VMEM((128,), jnp.float32) for _ in range(6)]
        + [pltpu.VMEM_SHARED((512,), jnp.float32)]
    )
    nms = functools.partial(
        pl.kernel,
        mesh=mesh,
        out_type=jax.ShapeDtypeStruct((_B, 6, 128), jnp.float32),
        scratch_types=scratch,
        compiler_params=pltpu.CompilerParams(needs_layout_passes=False),
    )(_nms_sc)
    out = nms(cand)
    return jnp.transpose(out[:, :, :_MAXDET], (0, 2, 1))


# speedup vs baseline: 1.7939x; 1.7939x over previous
import functools

import jax
import jax.numpy as jnp
from jax import lax
from jax.experimental import pallas as pl
from jax.experimental.pallas import tpu as pltpu
from jax.experimental.pallas import tpu_sc as plsc

_B = 4
_N = 5000
_NPAD = 5120
_BLKS = _NPAD // 128
_SEG = _NPAD // 8
_SEGCH = 40
_CAP = _SEG + 16
_MAXDET = 100
_MINCONF = 0.7
_NMS_T = 0.3
_BIG = jnp.int32(1 << 20)


def _prep_kernel(rois_ref, cls_ref, out_ref):
    y1 = rois_ref[:, 0]
    x1 = rois_ref[:, 1]
    y2 = rois_ref[:, 2]
    x2 = rois_ref[:, 3]
    dy = cls_ref[:, 0] * 0.1
    dx = cls_ref[:, 1] * 0.1
    dh = cls_ref[:, 2] * 0.2
    dw = cls_ref[:, 3] * 0.2
    cls_f = cls_ref[:, 4]
    raw_scores = cls_ref[:, 5]

    h = y2 - y1
    w = x2 - x1
    cy = y1 + 0.5 * h + dy * h
    cx = x1 + 0.5 * w + dx * w
    h = h * jnp.exp(dh)
    w = w * jnp.exp(dw)
    ry1 = jnp.clip(cy - 0.5 * h, 0.0, 1.0)
    rx1 = jnp.clip(cx - 0.5 * w, 0.0, 1.0)
    ry2 = jnp.clip((cy - 0.5 * h) + h, 0.0, 1.0)
    rx2 = jnp.clip((cx - 0.5 * w) + w, 0.0, 1.0)

    cls_i = cls_f.astype(jnp.int32)
    keep = (cls_i > 0) & (raw_scores >= _MINCONF)
    scores = jnp.where(keep, raw_scores, -1.0)

    off = cls_f * 4.0
    out_ref[:, 0] = ry1 + off
    out_ref[:, 1] = rx1 + off
    out_ref[:, 2] = ry2 + off
    out_ref[:, 3] = rx2 + off
    out_ref[:, 4] = cls_f
    out_ref[:, 5] = scores


def _top2_update(b1v, b1i, b2v, b2i, scm, posm):
    bet1 = (scm > b1v) | ((scm == b1v) & (posm < b1i))
    c2v = jnp.where(bet1, b1v, scm)
    c2i = jnp.where(bet1, b1i, posm)
    bet2 = (c2v > b2v) | ((c2v == b2v) & (c2i < b2i))
    return (
        jnp.where(bet1, scm, b1v),
        jnp.where(bet1, posm, b1i),
        jnp.where(bet2, c2v, b2v),
        jnp.where(bet2, c2i, b2i),
    )


def _xlane_top2(b1v, b1i, b2v, b2i):
    m1 = jnp.max(b1v)
    i1 = jnp.min(jnp.where(b1v == m1, b1i, _BIG))
    sel = (b1v == m1) & (b1i == i1)
    c2v = jnp.where(sel, b2v, b1v)
    c2i = jnp.where(sel, b2i, b1i)
    m2 = jnp.max(c2v)
    i2 = jnp.min(jnp.where(c2v == m2, c2i, _BIG))
    return m1, i1, m2, i2


def _nms_sc(cand_hbm, out_hbm, *refs):
    seg = refs[0:6]
    cch = refs[6:12]
    msg_v = refs[12]
    slots_v = refs[13]
    det = refs[14:20]
    slots_sh = refs[20]
    c = lax.axis_index("c")
    s = lax.axis_index("s")
    g = s // 8
    slot = s % 8
    batch = c * 2 + g
    row = g * 8 + slot
    iota = lax.iota(jnp.int32, 16)
    neg16 = jnp.full((16,), -1.0, jnp.float32)
    big16 = jnp.full((16,), _BIG)

    for k in range(6):
        pltpu.sync_copy(
            cand_hbm.at[batch, k, pl.ds(slot * _SEG, _SEG)], seg[k]
        )

    def compact_body(j, carry):
        cnt, b1v, b1i, b2v, b2i = carry
        idxv = j * 16 + iota
        sc = plsc.load_gather(seg[5], [idxv])
        m = sc > 0.0
        incl = plsc.cumsum(jnp.where(m, 1, 0))
        pos = cnt + incl - 1
        for k in range(5):
            v = plsc.load_gather(seg[k], [idxv])
            plsc.store_scatter(cch[k], [pos], v, mask=m)
        plsc.store_scatter(cch[5], [pos], sc, mask=m)
        scm = jnp.where(m, sc, -1.0)
        posm = jnp.where(m, pos, big16)
        b1v, b1i, b2v, b2i = _top2_update(b1v, b1i, b2v, b2i, scm, posm)
        return (cnt + jnp.max(incl), b1v, b1i, b2v, b2i)

    cnt, b1v0, b1i0, b2v0, b2i0 = lax.fori_loop(
        0, _SEGCH, compact_body, (jnp.int32(0), neg16, big16, neg16, big16)
    )
    plsc.store_scatter(
        cch[5], [cnt + iota], jnp.full((16,), -1.0, jnp.float32)
    )
    nchunks = (cnt + 15) // 16

    @pl.when(slot == 0)
    def _():
        for k in range(6):
            for j in range(8):
                det[k][pl.ds(j * 16, 16)] = jnp.zeros((16,), jnp.float32)

    def round_cond(carry):
        rnd, di, done_my, done_pub = carry[0], carry[1], carry[2], carry[3]
        return (rnd < _MAXDET + 4) & jnp.logical_not(done_pub)

    def round_body(carry):
        rnd, di, done_my, done_pub, b1v, b1i, b2v, b2i, npos = carry
        m1, i1, m2, i2 = _xlane_top2(b1v, b1i, b2v, b2i)
        iis1 = jnp.full((16,), jnp.minimum(i1, jnp.int32(_CAP - 1)), jnp.int32)
        iis2 = jnp.full((16,), jnp.minimum(i2, jnp.int32(_CAP - 1)), jnp.int32)
        has_more = npos > 2

        msg = jnp.where(iota == 5, m1, 0.0)
        msg = jnp.where(iota == 13, m2, msg)
        msg = jnp.where(iota == 6, jnp.where(done_my, 1.0, 0.0), msg)
        msg = jnp.where(iota == 7, jnp.where(has_more, 1.0, 0.0), msg)
        for k in range(5):
            v1 = plsc.load_gather(cch[k], [iis1])
            msg = jnp.where(iota == k, v1, msg)
            v2 = plsc.load_gather(cch[k], [iis2])
            msg = jnp.where(iota == k + 8, v2, msg)
        msg_v[...] = msg
        parity = rnd % 2
        pltpu.sync_copy(
            msg_v, slots_sh.at[pl.ds(parity * 256 + row * 16, 16)]
        )
        plsc.subcore_barrier()
        pltpu.sync_copy(slots_sh.at[pl.ds(parity * 256, 256)], slots_v)

        grow = plsc.load_gather(slots_v, [g * 128 + iota])
        my_done_pub = grow[6] > 0.5
        orow = plsc.load_gather(slots_v, [(1 - g) * 128 + iota])
        done_pub_new = my_done_pub & (orow[6] > 0.5)

        base = g * 128
        eoff = (iota // 2) * 16 + (iota % 2) * 8
        e_scv = plsc.load_gather(slots_v, [base + eoff + 5])
        hm_v = plsc.load_gather(slots_v, [base + (iota // 2) * 16 + 7]) > 0.5

        stopped = done_my
        exhausted = jnp.bool_(False)
        di_r = di
        own1 = jnp.bool_(False)
        own2 = jnp.bool_(False)
        sent = jnp.float32(1e9)
        av_y1 = jnp.full((16,), sent)
        av_x1 = jnp.full((16,), sent)
        av_y2 = jnp.full((16,), sent)
        av_x2 = jnp.full((16,), sent)
        av_ar = jnp.zeros((16,), jnp.float32)
        consumed_v = jnp.zeros((16,), jnp.int32)
        for k in range(8):
            blk = jnp.max(
                jnp.where((consumed_v >= 2) & hm_v, 1, 0)
            ) > 0
            stopped = stopped | blk

            esc = jnp.max(e_scv)
            eid = jnp.min(jnp.where(e_scv == esc, iota, _BIG))
            eid_safe = jnp.minimum(eid, jnp.int32(15))
            has_cand = esc > 0.0
            exhausted = exhausted | (
                jnp.logical_not(stopped) & jnp.logical_not(has_cand)
            )
            consider = jnp.logical_not(stopped) & has_cand & (
                di_r < _MAXDET
            )
            eslot = eid_safe // 2
            erank = eid_safe % 2
            crow = plsc.load_gather(
                slots_v,
                [jnp.full((16,), base + eslot * 16 + erank * 8, jnp.int32)
                 + iota],
            )
            cy1 = crow[0]
            cx1 = crow[1]
            cy2 = crow[2]
            cx2 = crow[3]
            ccls = crow[4]

            cy1v = jnp.full((16,), cy1)
            cx1v = jnp.full((16,), cx1)
            cy2v = jnp.full((16,), cy2)
            cx2v = jnp.full((16,), cx2)
            careav = (cy2v - cy1v) * (cx2v - cx1v)
            yy1 = jnp.maximum(av_y1, cy1v)
            xx1 = jnp.maximum(av_x1, cx1v)
            yy2 = jnp.minimum(av_y2, cy2v)
            xx2 = jnp.minimum(av_x2, cx2v)
            inter = jnp.maximum(yy2 - yy1, 0.0) * jnp.maximum(
                xx2 - xx1, 0.0
            )
            iou = inter / (av_ar + careav - inter + 1e-8)
            conflict = jnp.max(jnp.where(iou > _NMS_T, 1, 0)) > 0
            accept_k = consider & jnp.logical_not(conflict)
            consumed = consider
            own1 = own1 | (accept_k & (eslot == slot) & (erank == 0))
            own2 = own2 | (accept_k & (eslot == slot) & (erank == 1))

            @pl.when(accept_k & (slot == 0))
            def _(di_r=di_r, cy1=cy1, cx1=cx1, cy2=cy2, cx2=cx2,
                  ccls=ccls, esc=esc):
                hot = iota == 0
                dlane = jnp.full((16,), di_r, jnp.int32)
                offv = ccls * 4.0
                vals = (cy1 - offv, cx1 - offv, cy2 - offv, cx2 - offv,
                        ccls, esc)
                for kk in range(6):
                    plsc.store_scatter(
                        det[kk], [dlane], jnp.full((16,), vals[kk]),
                        mask=hot,
                    )

            lane_k = (iota == k) & accept_k
            av_y1 = jnp.where(lane_k, cy1v, av_y1)
            av_x1 = jnp.where(lane_k, cx1v, av_x1)
            av_y2 = jnp.where(lane_k, cy2v, av_y2)
            av_x2 = jnp.where(lane_k, cx2v, av_x2)
            av_ar = jnp.where(lane_k, careav, av_ar)
            di_r = di_r + jnp.where(accept_k, 1, 0)
            consumed_v = consumed_v + jnp.where(
                consumed & ((iota // 2) == eslot), 1, 0
            )
            e_scv = jnp.where(
                consumed & (iota == eid_safe), jnp.float32(-2.0), e_scv
            )

        done_my_new = done_my | exhausted | (di_r >= _MAXDET)

        plsc.store_scatter(cch[5], [iis1], neg16, mask=(iota == 0) & own1)
        plsc.store_scatter(cch[5], [iis2], neg16, mask=(iota == 0) & own2)

        wvec = [
            (
                jnp.full((16,), av_y1[k]),
                jnp.full((16,), av_x1[k]),
                jnp.full((16,), av_y2[k]),
                jnp.full((16,), av_x2[k]),
                jnp.full((16,), av_ar[k]),
            )
            for k in range(8)
        ]

        def sweep_body(j, carry2):
            nb1v, nb1i, nb2v, nb2i, cntv = carry2
            idxv = j * 16 + iota
            a0 = plsc.load_gather(cch[0], [idxv])
            a1 = plsc.load_gather(cch[1], [idxv])
            a2 = plsc.load_gather(cch[2], [idxv])
            a3 = plsc.load_gather(cch[3], [idxv])
            sc = plsc.load_gather(cch[5], [idxv])
            area = (a2 - a0) * (a3 - a1)
            for (wy1v, wx1v, wy2v, wx2v, wareav) in wvec:
                yy1 = jnp.maximum(wy1v, a0)
                xx1 = jnp.maximum(wx1v, a1)
                yy2 = jnp.minimum(wy2v, a2)
                xx2 = jnp.minimum(wx2v, a3)
                inter = jnp.maximum(yy2 - yy1, 0.0) * jnp.maximum(
                    xx2 - xx1, 0.0
                )
                iou = inter / (wareav + area - inter + 1e-8)
                sc = jnp.where(iou > _NMS_T, -1.0, sc)
            plsc.store_scatter(cch[5], [idxv], sc)
            nb1v, nb1i, nb2v, nb2i = _top2_update(
                nb1v, nb1i, nb2v, nb2i, sc, idxv
            )
            cntv = cntv + jnp.where(sc > 0.0, 1, 0)
            return nb1v, nb1i, nb2v, nb2i, cntv

        nb1v, nb1i, nb2v, nb2i, cntv = lax.fori_loop(
            0,
            nchunks,
            sweep_body,
            (neg16, big16, neg16, big16, jnp.zeros((16,), jnp.int32)),
        )
        npos_new = jnp.sum(cntv)

        return (
            rnd + 1,
            di_r,
            done_my_new,
            done_pub_new,
            nb1v,
            nb1i,
            nb2v,
            nb2i,
            npos_new,
        )

    lax.while_loop(
        round_cond,
        round_body,
        (
            jnp.int32(0),
            jnp.int32(0),
            jnp.bool_(False),
            jnp.bool_(False),
            b1v0,
            b1i0,
            b2v0,
            b2i0,
            cnt,
        ),
    )

    @pl.when(slot == 0)
    def _():
        for k in range(6):
            pltpu.sync_copy(det[k], out_hbm.at[batch, k])


def kernel(rois, classifications):
    rois_t = jnp.transpose(rois, (0, 2, 1))
    cls_t = jnp.transpose(classifications, (0, 2, 1))
    pad = _NPAD - _N
    rois_t = jnp.pad(rois_t, ((0, 0), (0, 0), (0, pad)))
    cls_t = jnp.pad(cls_t, ((0, 0), (0, 0), (0, pad)))
    rois_t = rois_t.reshape(_B, 4, _BLKS, 128)
    cls_t = cls_t.reshape(_B, 6, _BLKS, 128)

    cand = pl.pallas_call(
        _prep_kernel,
        out_shape=jax.ShapeDtypeStruct((_B, 6, _BLKS, 128), jnp.float32),
    )(rois_t, cls_t)
    cand = cand.reshape(_B, 6, _NPAD)

    mesh = plsc.VectorSubcoreMesh(core_axis_name="c", subcore_axis_name="s")
    scratch = (
        [pltpu.VMEM((_SEG,), jnp.float32) for _ in range(6)]
        + [pltpu.VMEM((_CAP,), jnp.float32) for _ in range(6)]
        + [pltpu.VMEM((16,), jnp.float32)]
        + [pltpu.VMEM((256,), jnp.float32)]
        + [pltpu.VMEM((128,), jnp.float32) for _ in range(6)]
        + [pltpu.VMEM_SHARED((512,), jnp.float32)]
    )
    nms = functools.partial(
        pl.kernel,
        mesh=mesh,
        out_type=jax.ShapeDtypeStruct((_B, 6, 128), jnp.float32),
        scratch_types=scratch,
        compiler_params=pltpu.CompilerParams(needs_layout_passes=False),
    )(_nms_sc)
    out = nms(cand)
    return jnp.transpose(out[:, :, :_MAXDET], (0, 2, 1))


# speedup vs baseline: 1.9484x; 1.0861x over previous
import functools

import jax
import jax.numpy as jnp
from jax import lax
from jax.experimental import pallas as pl
from jax.experimental.pallas import tpu as pltpu
from jax.experimental.pallas import tpu_sc as plsc

_B = 4
_N = 5000
_NPAD = 5120
_BLKS = _NPAD // 128
_SEG = _NPAD // 8
_SEGCH = 40
_CAP = _SEG + 16
_MAXDET = 100
_MINCONF = 0.7
_NMS_T = 0.3
_BIG = jnp.int32(1 << 20)


def _prep_kernel(rois_ref, cls_ref, out_ref):
    y1 = rois_ref[:, 0]
    x1 = rois_ref[:, 1]
    y2 = rois_ref[:, 2]
    x2 = rois_ref[:, 3]
    dy = cls_ref[:, 0] * 0.1
    dx = cls_ref[:, 1] * 0.1
    dh = cls_ref[:, 2] * 0.2
    dw = cls_ref[:, 3] * 0.2
    cls_f = cls_ref[:, 4]
    raw_scores = cls_ref[:, 5]

    h = y2 - y1
    w = x2 - x1
    cy = y1 + 0.5 * h + dy * h
    cx = x1 + 0.5 * w + dx * w
    h = h * jnp.exp(dh)
    w = w * jnp.exp(dw)
    ry1 = jnp.clip(cy - 0.5 * h, 0.0, 1.0)
    rx1 = jnp.clip(cx - 0.5 * w, 0.0, 1.0)
    ry2 = jnp.clip((cy - 0.5 * h) + h, 0.0, 1.0)
    rx2 = jnp.clip((cx - 0.5 * w) + w, 0.0, 1.0)

    cls_i = cls_f.astype(jnp.int32)
    keep = (cls_i > 0) & (raw_scores >= _MINCONF)
    scores = jnp.where(keep, raw_scores, -1.0)

    off = cls_f * 4.0
    out_ref[:, 0] = ry1 + off
    out_ref[:, 1] = rx1 + off
    out_ref[:, 2] = ry2 + off
    out_ref[:, 3] = rx2 + off
    out_ref[:, 4] = cls_f
    out_ref[:, 5] = scores


def _top2_update(b1v, b1i, b2v, b2i, scm, posm):
    bet1 = (scm > b1v) | ((scm == b1v) & (posm < b1i))
    c2v = jnp.where(bet1, b1v, scm)
    c2i = jnp.where(bet1, b1i, posm)
    bet2 = (c2v > b2v) | ((c2v == b2v) & (c2i < b2i))
    return (
        jnp.where(bet1, scm, b1v),
        jnp.where(bet1, posm, b1i),
        jnp.where(bet2, c2v, b2v),
        jnp.where(bet2, c2i, b2i),
    )


def _xlane_top2(b1v, b1i, b2v, b2i):
    m1 = jnp.max(b1v)
    i1 = jnp.min(jnp.where(b1v == m1, b1i, _BIG))
    sel = (b1v == m1) & (b1i == i1)
    c2v = jnp.where(sel, b2v, b1v)
    c2i = jnp.where(sel, b2i, b1i)
    m2 = jnp.max(c2v)
    i2 = jnp.min(jnp.where(c2v == m2, c2i, _BIG))
    return m1, i1, m2, i2


def _nms_sc(cand_hbm, out_hbm, *refs):
    seg = refs[0:6]
    cch = refs[6:12]
    msg_v = refs[12]
    slots_v = refs[13]
    det = refs[14:20]
    slots_sh = refs[20]
    c = lax.axis_index("c")
    s = lax.axis_index("s")
    g = s // 8
    slot = s % 8
    batch = c * 2 + g
    row = g * 8 + slot
    iota = lax.iota(jnp.int32, 16)
    neg16 = jnp.full((16,), -1.0, jnp.float32)
    big16 = jnp.full((16,), _BIG)

    for k in range(6):
        pltpu.sync_copy(
            cand_hbm.at[batch, k, pl.ds(slot * _SEG, _SEG)], seg[k]
        )

    def compact_body(j, carry):
        cnt, b1v, b1i, b2v, b2i = carry
        off = j * 16
        sc = seg[5][pl.ds(off, 16)]
        m = sc > 0.0
        incl = plsc.cumsum(jnp.where(m, 1, 0))
        pos = cnt + incl - 1
        for k in range(5):
            v = seg[k][pl.ds(off, 16)]
            plsc.store_scatter(cch[k], [pos], v, mask=m)
        plsc.store_scatter(cch[5], [pos], sc, mask=m)
        scm = jnp.where(m, sc, -1.0)
        posm = jnp.where(m, pos, big16)
        b1v, b1i, b2v, b2i = _top2_update(b1v, b1i, b2v, b2i, scm, posm)
        return (cnt + jnp.max(incl), b1v, b1i, b2v, b2i)

    cnt, b1v0, b1i0, b2v0, b2i0 = lax.fori_loop(
        0, _SEGCH, compact_body, (jnp.int32(0), neg16, big16, neg16, big16)
    )
    plsc.store_scatter(
        cch[5], [cnt + iota], jnp.full((16,), -1.0, jnp.float32)
    )
    nchunks = (cnt + 15) // 16

    @pl.when(slot == 0)
    def _():
        for k in range(6):
            for j in range(8):
                det[k][pl.ds(j * 16, 16)] = jnp.zeros((16,), jnp.float32)

    def round_cond(carry):
        rnd, di, done_my, done_pub = carry[0], carry[1], carry[2], carry[3]
        return (rnd < _MAXDET + 4) & jnp.logical_not(done_pub)

    def round_body(carry):
        rnd, di, done_my, done_pub, b1v, b1i, b2v, b2i, npos = carry
        m1, i1, m2, i2 = _xlane_top2(b1v, b1i, b2v, b2i)
        iis1 = jnp.full((16,), jnp.minimum(i1, jnp.int32(_CAP - 1)), jnp.int32)
        iis2 = jnp.full((16,), jnp.minimum(i2, jnp.int32(_CAP - 1)), jnp.int32)
        has_more = npos > 2

        msg = jnp.where(iota == 5, m1, 0.0)
        msg = jnp.where(iota == 13, m2, msg)
        msg = jnp.where(iota == 6, jnp.where(done_my, 1.0, 0.0), msg)
        msg = jnp.where(iota == 7, jnp.where(has_more, 1.0, 0.0), msg)
        for k in range(5):
            v1 = plsc.load_gather(cch[k], [iis1])
            msg = jnp.where(iota == k, v1, msg)
            v2 = plsc.load_gather(cch[k], [iis2])
            msg = jnp.where(iota == k + 8, v2, msg)
        msg_v[...] = msg
        parity = rnd % 2
        pltpu.sync_copy(
            msg_v, slots_sh.at[pl.ds(parity * 256 + row * 16, 16)]
        )
        plsc.subcore_barrier()
        pltpu.sync_copy(slots_sh.at[pl.ds(parity * 256, 256)], slots_v)

        grow = plsc.load_gather(slots_v, [g * 128 + iota])
        my_done_pub = grow[6] > 0.5
        orow = plsc.load_gather(slots_v, [(1 - g) * 128 + iota])
        done_pub_new = my_done_pub & (orow[6] > 0.5)

        base = g * 128
        eoff = (iota // 2) * 16 + (iota % 2) * 8
        e_scv = plsc.load_gather(slots_v, [base + eoff + 5])
        hm_v = plsc.load_gather(slots_v, [base + (iota // 2) * 16 + 7]) > 0.5

        stopped = done_my
        exhausted = jnp.bool_(False)
        di_r = di
        own1 = jnp.bool_(False)
        own2 = jnp.bool_(False)
        sent = jnp.float32(1e9)
        av_y1 = jnp.full((16,), sent)
        av_x1 = jnp.full((16,), sent)
        av_y2 = jnp.full((16,), sent)
        av_x2 = jnp.full((16,), sent)
        av_ar = jnp.zeros((16,), jnp.float32)
        consumed_v = jnp.zeros((16,), jnp.int32)
        for k in range(8):
            blk = jnp.max(
                jnp.where((consumed_v >= 2) & hm_v, 1, 0)
            ) > 0
            stopped = stopped | blk

            esc = jnp.max(e_scv)
            eid = jnp.min(jnp.where(e_scv == esc, iota, _BIG))
            eid_safe = jnp.minimum(eid, jnp.int32(15))
            has_cand = esc > 0.0
            exhausted = exhausted | (
                jnp.logical_not(stopped) & jnp.logical_not(has_cand)
            )
            consider = jnp.logical_not(stopped) & has_cand & (
                di_r < _MAXDET
            )
            eslot = eid_safe // 2
            erank = eid_safe % 2
            crow = plsc.load_gather(
                slots_v,
                [jnp.full((16,), base + eslot * 16 + erank * 8, jnp.int32)
                 + iota],
            )
            cy1 = crow[0]
            cx1 = crow[1]
            cy2 = crow[2]
            cx2 = crow[3]
            ccls = crow[4]

            cy1v = jnp.full((16,), cy1)
            cx1v = jnp.full((16,), cx1)
            cy2v = jnp.full((16,), cy2)
            cx2v = jnp.full((16,), cx2)
            careav = (cy2v - cy1v) * (cx2v - cx1v)
            yy1 = jnp.maximum(av_y1, cy1v)
            xx1 = jnp.maximum(av_x1, cx1v)
            yy2 = jnp.minimum(av_y2, cy2v)
            xx2 = jnp.minimum(av_x2, cx2v)
            inter = jnp.maximum(yy2 - yy1, 0.0) * jnp.maximum(
                xx2 - xx1, 0.0
            )
            iou = inter / (av_ar + careav - inter + 1e-8)
            conflict = jnp.max(jnp.where(iou > _NMS_T, 1, 0)) > 0
            accept_k = consider & jnp.logical_not(conflict)
            consumed = consider
            own1 = own1 | (accept_k & (eslot == slot) & (erank == 0))
            own2 = own2 | (accept_k & (eslot == slot) & (erank == 1))

            @pl.when(accept_k & (slot == 0))
            def _(di_r=di_r, cy1=cy1, cx1=cx1, cy2=cy2, cx2=cx2,
                  ccls=ccls, esc=esc):
                hot = iota == 0
                dlane = jnp.full((16,), di_r, jnp.int32)
                offv = ccls * 4.0
                vals = (cy1 - offv, cx1 - offv, cy2 - offv, cx2 - offv,
                        ccls, esc)
                for kk in range(6):
                    plsc.store_scatter(
                        det[kk], [dlane], jnp.full((16,), vals[kk]),
                        mask=hot,
                    )

            lane_k = (iota == k) & accept_k
            av_y1 = jnp.where(lane_k, cy1v, av_y1)
            av_x1 = jnp.where(lane_k, cx1v, av_x1)
            av_y2 = jnp.where(lane_k, cy2v, av_y2)
            av_x2 = jnp.where(lane_k, cx2v, av_x2)
            av_ar = jnp.where(lane_k, careav, av_ar)
            di_r = di_r + jnp.where(accept_k, 1, 0)
            consumed_v = consumed_v + jnp.where(
                consumed & ((iota // 2) == eslot), 1, 0
            )
            e_scv = jnp.where(
                consumed & (iota == eid_safe), jnp.float32(-2.0), e_scv
            )

        done_my_new = done_my | exhausted | (di_r >= _MAXDET)

        plsc.store_scatter(cch[5], [iis1], neg16, mask=(iota == 0) & own1)
        plsc.store_scatter(cch[5], [iis2], neg16, mask=(iota == 0) & own2)

        wvec = [
            (
                jnp.full((16,), av_y1[k]),
                jnp.full((16,), av_x1[k]),
                jnp.full((16,), av_y2[k]),
                jnp.full((16,), av_x2[k]),
                jnp.full((16,), av_ar[k]),
            )
            for k in range(8)
        ]

        def sweep_body(j, carry2):
            nb1v, nb1i, nb2v, nb2i, cntv = carry2
            idxv = j * 16 + iota
            off = j * 16
            a0 = cch[0][pl.ds(off, 16)]
            a1 = cch[1][pl.ds(off, 16)]
            a2 = cch[2][pl.ds(off, 16)]
            a3 = cch[3][pl.ds(off, 16)]
            sc = cch[5][pl.ds(off, 16)]
            area = (a2 - a0) * (a3 - a1)
            for (wy1v, wx1v, wy2v, wx2v, wareav) in wvec:
                yy1 = jnp.maximum(wy1v, a0)
                xx1 = jnp.maximum(wx1v, a1)
                yy2 = jnp.minimum(wy2v, a2)
                xx2 = jnp.minimum(wx2v, a3)
                inter = jnp.maximum(yy2 - yy1, 0.0) * jnp.maximum(
                    xx2 - xx1, 0.0
                )
                iou = inter / (wareav + area - inter + 1e-8)
                sc = jnp.where(iou > _NMS_T, -1.0, sc)
            cch[5][pl.ds(off, 16)] = sc
            nb1v, nb1i, nb2v, nb2i = _top2_update(
                nb1v, nb1i, nb2v, nb2i, sc, idxv
            )
            cntv = cntv + jnp.where(sc > 0.0, 1, 0)
            return nb1v, nb1i, nb2v, nb2i, cntv

        nb1v, nb1i, nb2v, nb2i, cntv = lax.fori_loop(
            0,
            nchunks,
            sweep_body,
            (neg16, big16, neg16, big16, jnp.zeros((16,), jnp.int32)),
        )
        npos_new = jnp.sum(cntv)

        return (
            rnd + 1,
            di_r,
            done_my_new,
            done_pub_new,
            nb1v,
            nb1i,
            nb2v,
            nb2i,
            npos_new,
        )

    lax.while_loop(
        round_cond,
        round_body,
        (
            jnp.int32(0),
            jnp.int32(0),
            jnp.bool_(False),
            jnp.bool_(False),
            b1v0,
            b1i0,
            b2v0,
            b2i0,
            cnt,
        ),
    )

    @pl.when(slot == 0)
    def _():
        for k in range(6):
            pltpu.sync_copy(det[k], out_hbm.at[batch, k])


def kernel(rois, classifications):
    rois_t = jnp.transpose(rois, (0, 2, 1))
    cls_t = jnp.transpose(classifications, (0, 2, 1))
    pad = _NPAD - _N
    rois_t = jnp.pad(rois_t, ((0, 0), (0, 0), (0, pad)))
    cls_t = jnp.pad(cls_t, ((0, 0), (0, 0), (0, pad)))
    rois_t = rois_t.reshape(_B, 4, _BLKS, 128)
    cls_t = cls_t.reshape(_B, 6, _BLKS, 128)

    cand = pl.pallas_call(
        _prep_kernel,
        out_shape=jax.ShapeDtypeStruct((_B, 6, _BLKS, 128), jnp.float32),
    )(rois_t, cls_t)
    cand = cand.reshape(_B, 6, _NPAD)

    mesh = plsc.VectorSubcoreMesh(core_axis_name="c", subcore_axis_name="s")
    scratch = (
        [pltpu.VMEM((_SEG,), jnp.float32) for _ in range(6)]
        + [pltpu.VMEM((_CAP,), jnp.float32) for _ in range(6)]
        + [pltpu.VMEM((16,), jnp.float32)]
        + [pltpu.VMEM((256,), jnp.float32)]
        + [pltpu.VMEM((128,), jnp.float32) for _ in range(6)]
        + [pltpu.VMEM_SHARED((512,), jnp.float32)]
    )
    nms = functools.partial(
        pl.kernel,
        mesh=mesh,
        out_type=jax.ShapeDtypeStruct((_B, 6, 128), jnp.float32),
        scratch_types=scratch,
        compiler_params=pltpu.CompilerParams(needs_layout_passes=False),
    )(_nms_sc)
    out = nms(cand)
    return jnp.transpose(out[:, :, :_MAXDET], (0, 2, 1))


# speedup vs baseline: 2.2554x; 1.1576x over previous
import functools

import jax
import jax.numpy as jnp
from jax import lax
from jax.experimental import pallas as pl
from jax.experimental.pallas import tpu as pltpu
from jax.experimental.pallas import tpu_sc as plsc

_B = 4
_N = 5000
_NPAD = 5120
_BLKS = _NPAD // 128
_SEG = _NPAD // 8
_SEGCH = 40
_CAP = _SEG + 16
_MAXDET = 100
_MINCONF = 0.7
_NMS_T = 0.3
_BIG = jnp.int32(1 << 20)


def _prep_kernel(rois_ref, cls_ref, out_ref):
    y1 = rois_ref[:, 0]
    x1 = rois_ref[:, 1]
    y2 = rois_ref[:, 2]
    x2 = rois_ref[:, 3]
    dy = cls_ref[:, 0] * 0.1
    dx = cls_ref[:, 1] * 0.1
    dh = cls_ref[:, 2] * 0.2
    dw = cls_ref[:, 3] * 0.2
    cls_f = cls_ref[:, 4]
    raw_scores = cls_ref[:, 5]

    h = y2 - y1
    w = x2 - x1
    cy = y1 + 0.5 * h + dy * h
    cx = x1 + 0.5 * w + dx * w
    h = h * jnp.exp(dh)
    w = w * jnp.exp(dw)
    ry1 = jnp.clip(cy - 0.5 * h, 0.0, 1.0)
    rx1 = jnp.clip(cx - 0.5 * w, 0.0, 1.0)
    ry2 = jnp.clip((cy - 0.5 * h) + h, 0.0, 1.0)
    rx2 = jnp.clip((cx - 0.5 * w) + w, 0.0, 1.0)

    cls_i = cls_f.astype(jnp.int32)
    keep = (cls_i > 0) & (raw_scores >= _MINCONF)
    scores = jnp.where(keep, raw_scores, -1.0)

    off = cls_f * 4.0
    out_ref[:, 0] = ry1 + off
    out_ref[:, 1] = rx1 + off
    out_ref[:, 2] = ry2 + off
    out_ref[:, 3] = rx2 + off
    out_ref[:, 4] = cls_f
    out_ref[:, 5] = scores


def _top2_update(b1v, b1i, b2v, b2i, scm, posm):
    bet1 = (scm > b1v) | ((scm == b1v) & (posm < b1i))
    c2v = jnp.where(bet1, b1v, scm)
    c2i = jnp.where(bet1, b1i, posm)
    bet2 = (c2v > b2v) | ((c2v == b2v) & (c2i < b2i))
    return (
        jnp.where(bet1, scm, b1v),
        jnp.where(bet1, posm, b1i),
        jnp.where(bet2, c2v, b2v),
        jnp.where(bet2, c2i, b2i),
    )


def _xlane_top2(b1v, b1i, b2v, b2i):
    m1 = jnp.max(b1v)
    i1 = jnp.min(jnp.where(b1v == m1, b1i, _BIG))
    sel = (b1v == m1) & (b1i == i1)
    c2v = jnp.where(sel, b2v, b1v)
    c2i = jnp.where(sel, b2i, b1i)
    m2 = jnp.max(c2v)
    i2 = jnp.min(jnp.where(c2v == m2, c2i, _BIG))
    return m1, i1, m2, i2


def _nms_sc(cand_hbm, out_hbm, *refs):
    seg = refs[0:6]
    cch = refs[6:12]
    msg_v = refs[12]
    slots_v = refs[13]
    det = refs[14:20]
    slots_sh = refs[20]
    c = lax.axis_index("c")
    s = lax.axis_index("s")
    g = s // 8
    slot = s % 8
    batch = c * 2 + g
    row = g * 8 + slot
    iota = lax.iota(jnp.int32, 16)
    neg16 = jnp.full((16,), -1.0, jnp.float32)
    big16 = jnp.full((16,), _BIG)

    for k in range(6):
        pltpu.sync_copy(
            cand_hbm.at[batch, k, pl.ds(slot * _SEG, _SEG)], seg[k]
        )

    def compact_body(j, carry):
        cnt, b1v, b1i, b2v, b2i = carry
        off = j * 16
        sc = seg[5][pl.ds(off, 16)]
        m = sc > 0.0
        incl = plsc.cumsum(jnp.where(m, 1, 0))
        pos = cnt + incl - 1
        for k in range(5):
            v = seg[k][pl.ds(off, 16)]
            plsc.store_scatter(cch[k], [pos], v, mask=m)
        plsc.store_scatter(cch[5], [pos], sc, mask=m)
        scm = jnp.where(m, sc, -1.0)
        posm = jnp.where(m, pos, big16)
        b1v, b1i, b2v, b2i = _top2_update(b1v, b1i, b2v, b2i, scm, posm)
        return (cnt + jnp.max(incl), b1v, b1i, b2v, b2i)

    cnt, b1v0, b1i0, b2v0, b2i0 = lax.fori_loop(
        0, _SEGCH, compact_body, (jnp.int32(0), neg16, big16, neg16, big16)
    )
    plsc.store_scatter(
        cch[5], [cnt + iota], jnp.full((16,), -1.0, jnp.float32)
    )
    nchunks = (cnt + 15) // 16

    @pl.when(slot == 0)
    def _():
        for k in range(6):
            for j in range(8):
                det[k][pl.ds(j * 16, 16)] = jnp.zeros((16,), jnp.float32)

    def round_cond(carry):
        rnd, di, done_my, done_pub = carry[0], carry[1], carry[2], carry[3]
        return (rnd < _MAXDET + 4) & jnp.logical_not(done_pub)

    def round_body(carry):
        rnd, di, done_my, done_pub, b1v, b1i, b2v, b2i, npos = carry
        m1, i1, m2, i2 = _xlane_top2(b1v, b1i, b2v, b2i)
        iis1 = jnp.full((16,), jnp.minimum(i1, jnp.int32(_CAP - 1)), jnp.int32)
        iis2 = jnp.full((16,), jnp.minimum(i2, jnp.int32(_CAP - 1)), jnp.int32)
        has_more = npos > 2

        msg = jnp.where(iota == 5, m1, 0.0)
        msg = jnp.where(iota == 13, m2, msg)
        msg = jnp.where(iota == 6, jnp.where(done_my, 1.0, 0.0), msg)
        msg = jnp.where(iota == 7, jnp.where(has_more, 1.0, 0.0), msg)
        for k in range(5):
            v1 = plsc.load_gather(cch[k], [iis1])
            msg = jnp.where(iota == k, v1, msg)
            v2 = plsc.load_gather(cch[k], [iis2])
            msg = jnp.where(iota == k + 8, v2, msg)
        msg_v[...] = msg
        parity = rnd % 2
        pltpu.sync_copy(
            msg_v, slots_sh.at[pl.ds(parity * 256 + row * 16, 16)]
        )
        plsc.subcore_barrier()
        pltpu.sync_copy(slots_sh.at[pl.ds(parity * 256, 256)], slots_v)

        grow = plsc.load_gather(slots_v, [g * 128 + iota])
        my_done_pub = grow[6] > 0.5
        orow = plsc.load_gather(slots_v, [(1 - g) * 128 + iota])
        done_pub_new = my_done_pub & (orow[6] > 0.5)

        base = g * 128
        eoff = (iota // 2) * 16 + (iota % 2) * 8
        e_scv = plsc.load_gather(slots_v, [base + eoff + 5])
        hm_v = plsc.load_gather(slots_v, [base + (iota // 2) * 16 + 7]) > 0.5

        stopped = done_my
        exhausted = jnp.bool_(False)
        di_r = di
        own1 = jnp.bool_(False)
        own2 = jnp.bool_(False)
        sent = jnp.float32(1e9)
        av_y1 = jnp.full((16,), sent)
        av_x1 = jnp.full((16,), sent)
        av_y2 = jnp.full((16,), sent)
        av_x2 = jnp.full((16,), sent)
        av_ar = jnp.zeros((16,), jnp.float32)
        consumed_v = jnp.zeros((16,), jnp.int32)
        for k in range(4):
            blk = jnp.max(
                jnp.where((consumed_v >= 2) & hm_v, 1, 0)
            ) > 0
            stopped = stopped | blk

            esc = jnp.max(e_scv)
            eid = jnp.min(jnp.where(e_scv == esc, iota, _BIG))
            eid_safe = jnp.minimum(eid, jnp.int32(15))
            has_cand = esc > 0.0
            exhausted = exhausted | (
                jnp.logical_not(stopped) & jnp.logical_not(has_cand)
            )
            consider = jnp.logical_not(stopped) & has_cand & (
                di_r < _MAXDET
            )
            eslot = eid_safe // 2
            erank = eid_safe % 2
            crow = plsc.load_gather(
                slots_v,
                [jnp.full((16,), base + eslot * 16 + erank * 8, jnp.int32)
                 + iota],
            )
            cy1 = crow[0]
            cx1 = crow[1]
            cy2 = crow[2]
            cx2 = crow[3]
            ccls = crow[4]

            cy1v = jnp.full((16,), cy1)
            cx1v = jnp.full((16,), cx1)
            cy2v = jnp.full((16,), cy2)
            cx2v = jnp.full((16,), cx2)
            careav = (cy2v - cy1v) * (cx2v - cx1v)
            yy1 = jnp.maximum(av_y1, cy1v)
            xx1 = jnp.maximum(av_x1, cx1v)
            yy2 = jnp.minimum(av_y2, cy2v)
            xx2 = jnp.minimum(av_x2, cx2v)
            inter = jnp.maximum(yy2 - yy1, 0.0) * jnp.maximum(
                xx2 - xx1, 0.0
            )
            iou = inter / (av_ar + careav - inter + 1e-8)
            conflict = jnp.max(jnp.where(iou > _NMS_T, 1, 0)) > 0
            accept_k = consider & jnp.logical_not(conflict)
            consumed = consider
            own1 = own1 | (accept_k & (eslot == slot) & (erank == 0))
            own2 = own2 | (accept_k & (eslot == slot) & (erank == 1))

            @pl.when(accept_k & (slot == 0))
            def _(di_r=di_r, cy1=cy1, cx1=cx1, cy2=cy2, cx2=cx2,
                  ccls=ccls, esc=esc):
                hot = iota == 0
                dlane = jnp.full((16,), di_r, jnp.int32)
                offv = ccls * 4.0
                vals = (cy1 - offv, cx1 - offv, cy2 - offv, cx2 - offv,
                        ccls, esc)
                for kk in range(6):
                    plsc.store_scatter(
                        det[kk], [dlane], jnp.full((16,), vals[kk]),
                        mask=hot,
                    )

            lane_k = (iota == k) & accept_k
            av_y1 = jnp.where(lane_k, cy1v, av_y1)
            av_x1 = jnp.where(lane_k, cx1v, av_x1)
            av_y2 = jnp.where(lane_k, cy2v, av_y2)
            av_x2 = jnp.where(lane_k, cx2v, av_x2)
            av_ar = jnp.where(lane_k, careav, av_ar)
            di_r = di_r + jnp.where(accept_k, 1, 0)
            consumed_v = consumed_v + jnp.where(
                consumed & ((iota // 2) == eslot), 1, 0
            )
            e_scv = jnp.where(
                consumed & (iota == eid_safe), jnp.float32(-2.0), e_scv
            )

        done_my_new = done_my | exhausted | (di_r >= _MAXDET)

        plsc.store_scatter(cch[5], [iis1], neg16, mask=(iota == 0) & own1)
        plsc.store_scatter(cch[5], [iis2], neg16, mask=(iota == 0) & own2)

        wvec = [
            (
                jnp.full((16,), av_y1[k]),
                jnp.full((16,), av_x1[k]),
                jnp.full((16,), av_y2[k]),
                jnp.full((16,), av_x2[k]),
                jnp.full((16,), av_ar[k]),
            )
            for k in range(4)
        ]

        def sweep_body(j, carry2):
            nb1v, nb1i, nb2v, nb2i, cntv = carry2
            idxv = j * 16 + iota
            off = j * 16
            a0 = cch[0][pl.ds(off, 16)]
            a1 = cch[1][pl.ds(off, 16)]
            a2 = cch[2][pl.ds(off, 16)]
            a3 = cch[3][pl.ds(off, 16)]
            sc = cch[5][pl.ds(off, 16)]
            area = (a2 - a0) * (a3 - a1)
            for (wy1v, wx1v, wy2v, wx2v, wareav) in wvec:
                yy1 = jnp.maximum(wy1v, a0)
                xx1 = jnp.maximum(wx1v, a1)
                yy2 = jnp.minimum(wy2v, a2)
                xx2 = jnp.minimum(wx2v, a3)
                inter = jnp.maximum(yy2 - yy1, 0.0) * jnp.maximum(
                    xx2 - xx1, 0.0
                )
                iou = inter / (wareav + area - inter + 1e-8)
                sc = jnp.where(iou > _NMS_T, -1.0, sc)
            cch[5][pl.ds(off, 16)] = sc
            nb1v, nb1i, nb2v, nb2i = _top2_update(
                nb1v, nb1i, nb2v, nb2i, sc, idxv
            )
            cntv = cntv + jnp.where(sc > 0.0, 1, 0)
            return nb1v, nb1i, nb2v, nb2i, cntv

        nb1v, nb1i, nb2v, nb2i, cntv = lax.fori_loop(
            0,
            nchunks,
            sweep_body,
            (neg16, big16, neg16, big16, jnp.zeros((16,), jnp.int32)),
        )
        npos_new = jnp.sum(cntv)

        return (
            rnd + 1,
            di_r,
            done_my_new,
            done_pub_new,
            nb1v,
            nb1i,
            nb2v,
            nb2i,
            npos_new,
        )

    lax.while_loop(
        round_cond,
        round_body,
        (
            jnp.int32(0),
            jnp.int32(0),
            jnp.bool_(False),
            jnp.bool_(False),
            b1v0,
            b1i0,
            b2v0,
            b2i0,
            cnt,
        ),
    )

    @pl.when(slot == 0)
    def _():
        for k in range(6):
            pltpu.sync_copy(det[k], out_hbm.at[batch, k])


def kernel(rois, classifications):
    rois_t = jnp.transpose(rois, (0, 2, 1))
    cls_t = jnp.transpose(classifications, (0, 2, 1))
    pad = _NPAD - _N
    rois_t = jnp.pad(rois_t, ((0, 0), (0, 0), (0, pad)))
    cls_t = jnp.pad(cls_t, ((0, 0), (0, 0), (0, pad)))
    rois_t = rois_t.reshape(_B, 4, _BLKS, 128)
    cls_t = cls_t.reshape(_B, 6, _BLKS, 128)

    cand = pl.pallas_call(
        _prep_kernel,
        out_shape=jax.ShapeDtypeStruct((_B, 6, _BLKS, 128), jnp.float32),
    )(rois_t, cls_t)
    cand = cand.reshape(_B, 6, _NPAD)

    mesh = plsc.VectorSubcoreMesh(core_axis_name="c", subcore_axis_name="s")
    scratch = (
        [pltpu.VMEM((_SEG,), jnp.float32) for _ in range(6)]
        + [pltpu.VMEM((_CAP,), jnp.float32) for _ in range(6)]
        + [pltpu.VMEM((16,), jnp.float32)]
        + [pltpu.VMEM((256,), jnp.float32)]
        + [pltpu.VMEM((128,), jnp.float32) for _ in range(6)]
        + [pltpu.VMEM_SHARED((512,), jnp.float32)]
    )
    nms = functools.partial(
        pl.kernel,
        mesh=mesh,
        out_type=jax.ShapeDtypeStruct((_B, 6, 128), jnp.float32),
        scratch_types=scratch,
        compiler_params=pltpu.CompilerParams(needs_layout_passes=False),
    )(_nms_sc)
    out = nms(cand)
    return jnp.transpose(out[:, :, :_MAXDET], (0, 2, 1))
